# baseline (speedup 1.0000x reference)
import functools

import jax
import jax.numpy as jnp
from jax import lax
from jax.experimental import pallas as pl
from jax.experimental.pallas import tpu as pltpu
from jax.experimental.pallas import tpu_sc as plsc

NC = 2
NS = 16
NW = NC * NS
L = 16
CH = 128
GRP = 8
NBG = 5
GAH = 3


def _mesh():
    return plsc.VectorSubcoreMesh(
        core_axis_name="c", subcore_axis_name="s", num_cores=NC, num_subcores=NS)


_SC_PARAMS = pltpu.CompilerParams(use_tc_tiling_on_sc=False)


GC = GRP * CH


def _load_edge_row(ei_hbm, row, idx_v, wid, epw0, nfull, rem, n, lsem):
    descs = []
    base = wid * epw0
    for r in range(nfull):
        descs.append(pltpu.async_copy(
            ei_hbm.at[row, pl.ds(base + r * GC, GC)], idx_v.at[r], lsem))
    if rem:
        iota16 = lax.iota(jnp.int32, L)
        for k in range((GC - rem) // L):
            idx_v[nfull, pl.ds(rem + k * L, L)] = n + ((iota16 + k * L) & 127)
        descs.append(pltpu.async_copy(
            ei_hbm.at[row, pl.ds(base + nfull * GC, rem)],
            idx_v.at[nfull, pl.ds(0, rem)], lsem))
    return descs


def _deg_kernel(ngrp, epw0, nfull, rem, n, n_pad):
    rpt = n_pad // NS

    @functools.partial(
        pl.kernel,
        out_type=jax.ShapeDtypeStruct((NC, n_pad), jnp.float32),
        mesh=_mesh(),
        scratch_types=[
            pltpu.VMEM((ngrp, GC), jnp.int32),
            pltpu.VMEM((GC,), jnp.float32),
            pltpu.VMEM((rpt,), jnp.float32),
            pltpu.VMEM_SHARED((n_pad,), jnp.float32),
            pltpu.SemaphoreType.DMA,
            pltpu.SemaphoreType.DMA,
        ],
        compiler_params=_SC_PARAMS,
    )
    def deg_k(ei_hbm, degp_hbm, idx_v, ones_v, stage_v, deg_sh, sem, lsem):
        cid = lax.axis_index("c")
        tid = lax.axis_index("s")
        wid = cid * NS + tid
        r0 = tid * rpt
        one16 = jnp.ones((L,), jnp.float32)
        zero16 = jnp.zeros((L,), jnp.float32)
        ld = _load_edge_row(ei_hbm, 1, idx_v, wid, epw0, nfull, rem, n, lsem)
        for k in range(GC // L):
            ones_v[pl.ds(k * L, L)] = one16
        for k in range(rpt // L):
            stage_v[pl.ds(k * L, L)] = zero16
        pltpu.sync_copy(stage_v, deg_sh.at[pl.ds(r0, rpt)])
        for dsc in ld:
            dsc.wait()
        plsc.subcore_barrier()
        descs = [None] * ngrp
        for j in range(ngrp):
            descs[j] = pltpu.async_copy(
                ones_v, deg_sh.at[idx_v.at[j]], sem, add=True)
            if j >= 4:
                descs[j - 4].wait()
        for j in range(max(0, ngrp - 4), ngrp):
            descs[j].wait()
        plsc.subcore_barrier()
        pltpu.sync_copy(deg_sh.at[pl.ds(r0, rpt)], degp_hbm.at[cid, pl.ds(r0, rpt)])

    return deg_k


def _msg_kernel(ngrp, epw0, nfull, rem, n, n_pad, dh):
    rpt = n_pad // NS

    @functools.partial(
        pl.kernel,
        out_type=jax.ShapeDtypeStruct((NC, n_pad, dh), jnp.float32),
        mesh=_mesh(),
        scratch_types=[
            pltpu.VMEM((ngrp, GC), jnp.int32),
            pltpu.VMEM((ngrp, GC), jnp.int32),
            pltpu.VMEM((NBG, GC, dh), jnp.float32),
            pltpu.VMEM((rpt, dh), jnp.float32),
            pltpu.VMEM_SHARED((n_pad, dh), jnp.float32),
            pltpu.SemaphoreType.DMA,
            pltpu.SemaphoreType.DMA,
            pltpu.SemaphoreType.DMA,
        ],
        compiler_params=_SC_PARAMS,
    )
    def msg_k(g_hbm, ei_hbm, zero_hbm, sp_hbm,
              src_v, dst_v, msg_v, stage_v, s_sh, gsem, ssem, lsem):
        cid = lax.axis_index("c")
        tid = lax.axis_index("s")
        wid = cid * NS + tid
        r0 = tid * rpt
        ld = _load_edge_row(ei_hbm, 0, src_v, wid, epw0, nfull, rem, n, lsem)
        ld += _load_edge_row(ei_hbm, 1, dst_v, wid, epw0, nfull, rem, n, lsem)
        pltpu.sync_copy(zero_hbm.at[pl.ds(r0, rpt)], s_sh.at[pl.ds(r0, rpt)])
        for dsc in ld:
            dsc.wait()
        plsc.subcore_barrier()
        gd = [None] * ngrp
        sd = [None] * ngrp
        for j in range(min(GAH, ngrp)):
            gd[j] = pltpu.async_copy(
                g_hbm.at[src_v.at[j]], msg_v.at[j % NBG], gsem)
        for j in range(ngrp):
            b = j % NBG
            gd[j].wait()
            sd[j] = pltpu.async_copy(
                msg_v.at[b], s_sh.at[dst_v.at[j]], ssem, add=True)
            nj = j + GAH
            if nj < ngrp:
                if nj >= NBG:
                    sd[nj - NBG].wait()
                gd[nj] = pltpu.async_copy(
                    g_hbm.at[src_v.at[nj]], msg_v.at[nj % NBG], gsem)
        for j in range(max(0, ngrp - NBG), ngrp):
            sd[j].wait()
        plsc.subcore_barrier()
        pltpu.sync_copy(s_sh.at[pl.ds(r0, rpt)], sp_hbm.at[cid, pl.ds(r0, rpt)])

    return msg_k


def _encode_body(x_ref, w_ref, degp_ref, g_ref, dr_ref):
    deg = degp_ref[0:1, :] + degp_ref[1:2, :] + 1.0
    dinv_row = lax.rsqrt(deg)
    dinv = jnp.transpose(dinv_row)
    h = jnp.dot(x_ref[...], w_ref[...], preferred_element_type=jnp.float32,
                precision=lax.Precision.HIGHEST)
    g_ref[...] = h * dinv
    dr_ref[...] = dinv_row


def _z_body(sp_ref, g_ref, dr_ref, b_ref, z_ref):
    s = sp_ref[0] + sp_ref[1] + g_ref[...]
    dinv = jnp.transpose(dr_ref[...])
    z_ref[...] = jnp.maximum(dinv * s + b_ref[...], 0.0)


def _decode_body(bmr, bnc, z_ref, out_ref):
    i = pl.program_id(0)
    j = pl.program_id(1)
    zr = z_ref[pl.ds(i * bmr, bmr), :]
    zc = z_ref[pl.ds(j * bnc, bnc), :]
    logits = lax.dot_general(
        zr, zc, (((1,), (1,)), ((), ())),
        preferred_element_type=jnp.float32)
    out_ref[...] = 0.5 * jnp.tanh(0.5 * logits) + 0.5


def kernel(x, edge_index, W, b):
    n, d_in = x.shape
    dh = W.shape[1]
    e = edge_index.shape[1]

    n_pad = ((n + 128 + 255) // 256) * 256
    assert e % (NW * L) == 0
    epw0 = e // NW
    nfull, rem = divmod(epw0, GC)
    ngrp = nfull + (1 if rem else 0)
    ei32 = edge_index.astype(jnp.int32)

    degp = _deg_kernel(ngrp, epw0, nfull, rem, n, n_pad)(ei32)

    bm = 2048
    g_pad, dinv_row = pl.pallas_call(
        _encode_body,
        grid=(n_pad // bm,),
        in_specs=[
            pl.BlockSpec((bm, d_in), lambda i: (i, 0)),
            pl.BlockSpec((d_in, dh), lambda i: (0, 0)),
            pl.BlockSpec((2, bm), lambda i: (0, i)),
        ],
        out_specs=[
            pl.BlockSpec((bm, dh), lambda i: (i, 0)),
            pl.BlockSpec((1, bm), lambda i: (0, i)),
        ],
        out_shape=[
            jax.ShapeDtypeStruct((n_pad, dh), jnp.float32),
            jax.ShapeDtypeStruct((1, n_pad), jnp.float32),
        ],
    )(x, W, degp)

    zeros2d = jnp.zeros((n_pad, dh), jnp.float32)
    sp = _msg_kernel(ngrp, epw0, nfull, rem, n, n_pad, dh)(g_pad, ei32, zeros2d)

    z_pad = pl.pallas_call(
        _z_body,
        grid=(n_pad // bm,),
        in_specs=[
            pl.BlockSpec((2, bm, dh), lambda i: (0, i, 0)),
            pl.BlockSpec((bm, dh), lambda i: (i, 0)),
            pl.BlockSpec((1, bm), lambda i: (0, i)),
            pl.BlockSpec((1, dh), lambda i: (0, 0)),
        ],
        out_specs=pl.BlockSpec((bm, dh), lambda i: (i, 0)),
        out_shape=jax.ShapeDtypeStruct((n_pad, dh), jnp.float32),
    )(sp, g_pad, dinv_row, b.reshape(1, dh))

    bmr, bnc = 2048, 2048
    adj = pl.pallas_call(
        functools.partial(_decode_body, bmr, bnc),
        grid=(-(-n // bmr), -(-n // bnc)),
        in_specs=[
            pl.BlockSpec((n_pad, dh), lambda i, j: (0, 0)),
        ],
        out_specs=pl.BlockSpec((bmr, bnc), lambda i, j: (i, j)),
        out_shape=jax.ShapeDtypeStruct((n, n), jnp.float32),
    )(z_pad)
    return adj


# speedup vs baseline: 1.0027x; 1.0027x over previous
import functools

import jax
import jax.numpy as jnp
from jax import lax
from jax.experimental import pallas as pl
from jax.experimental.pallas import tpu as pltpu
from jax.experimental.pallas import tpu_sc as plsc

NC = 2
NS = 16
NW = NC * NS
L = 16
CH = 128
GRP = 8
NBG = 5
GAH = 3


def _mesh():
    return plsc.VectorSubcoreMesh(
        core_axis_name="c", subcore_axis_name="s", num_cores=NC, num_subcores=NS)


_SC_PARAMS = pltpu.CompilerParams(use_tc_tiling_on_sc=False)


GC = GRP * CH


def _load_edge_row(ei_hbm, row, idx_v, wid, epw0, nfull, rem, n, lsem):
    descs = []
    base = wid * epw0
    for r in range(nfull):
        descs.append(pltpu.async_copy(
            ei_hbm.at[row, pl.ds(base + r * GC, GC)], idx_v.at[r], lsem))
    if rem:
        iota16 = lax.iota(jnp.int32, L)
        for k in range((GC - rem) // L):
            idx_v[nfull, pl.ds(rem + k * L, L)] = n + ((iota16 + k * L) & 127)
        descs.append(pltpu.async_copy(
            ei_hbm.at[row, pl.ds(base + nfull * GC, rem)],
            idx_v.at[nfull, pl.ds(0, rem)], lsem))
    return descs


def _deg_kernel(ngrp, epw0, nfull, rem, n, n_pad):
    rpt = n_pad // NS

    @functools.partial(
        pl.kernel,
        out_type=jax.ShapeDtypeStruct((NC, n_pad), jnp.float32),
        mesh=_mesh(),
        scratch_types=[
            pltpu.VMEM((ngrp, GC), jnp.int32),
            pltpu.VMEM((GC,), jnp.float32),
            pltpu.VMEM((rpt,), jnp.float32),
            pltpu.VMEM_SHARED((n_pad,), jnp.float32),
            pltpu.SemaphoreType.DMA,
            pltpu.SemaphoreType.DMA,
        ],
        compiler_params=_SC_PARAMS,
    )
    def deg_k(ei_hbm, degp_hbm, idx_v, ones_v, stage_v, deg_sh, sem, lsem):
        cid = lax.axis_index("c")
        tid = lax.axis_index("s")
        wid = cid * NS + tid
        r0 = tid * rpt
        one16 = jnp.ones((L,), jnp.float32)
        zero16 = jnp.zeros((L,), jnp.float32)
        ld = _load_edge_row(ei_hbm, 1, idx_v, wid, epw0, nfull, rem, n, lsem)
        for k in range(GC // L):
            ones_v[pl.ds(k * L, L)] = one16
        for k in range(rpt // L):
            stage_v[pl.ds(k * L, L)] = zero16
        pltpu.sync_copy(stage_v, deg_sh.at[pl.ds(r0, rpt)])
        for dsc in ld:
            dsc.wait()
        plsc.subcore_barrier()
        descs = [None] * ngrp
        for j in range(ngrp):
            descs[j] = pltpu.async_copy(
                ones_v, deg_sh.at[idx_v.at[j]], sem, add=True)
            if j >= 4:
                descs[j - 4].wait()
        for j in range(max(0, ngrp - 4), ngrp):
            descs[j].wait()
        plsc.subcore_barrier()
        pltpu.sync_copy(deg_sh.at[pl.ds(r0, rpt)], degp_hbm.at[cid, pl.ds(r0, rpt)])

    return deg_k


def _msg_kernel(ngrp, epw0, nfull, rem, n, n_pad, dh):
    rpt = n_pad // NS

    @functools.partial(
        pl.kernel,
        out_type=jax.ShapeDtypeStruct((NC, n_pad, dh), jnp.float32),
        mesh=_mesh(),
        scratch_types=[
            pltpu.VMEM((ngrp, GC), jnp.int32),
            pltpu.VMEM((ngrp, GC), jnp.int32),
            pltpu.VMEM((NBG, GC, dh), jnp.float32),
            pltpu.VMEM((rpt, dh), jnp.float32),
            pltpu.VMEM_SHARED((n_pad, dh), jnp.float32),
            pltpu.SemaphoreType.DMA,
            pltpu.SemaphoreType.DMA,
            pltpu.SemaphoreType.DMA,
        ],
        compiler_params=_SC_PARAMS,
    )
    def msg_k(g_hbm, ei_hbm, zero_hbm, sp_hbm,
              src_v, dst_v, msg_v, stage_v, s_sh, gsem, ssem, lsem):
        cid = lax.axis_index("c")
        tid = lax.axis_index("s")
        wid = cid * NS + tid
        r0 = tid * rpt
        ld = _load_edge_row(ei_hbm, 0, src_v, wid, epw0, nfull, rem, n, lsem)
        ld += _load_edge_row(ei_hbm, 1, dst_v, wid, epw0, nfull, rem, n, lsem)
        pltpu.sync_copy(zero_hbm.at[pl.ds(r0, rpt)], s_sh.at[pl.ds(r0, rpt)])
        for dsc in ld:
            dsc.wait()
        plsc.subcore_barrier()
        gd = [None] * ngrp
        sd = [None] * ngrp
        for j in range(min(GAH, ngrp)):
            gd[j] = pltpu.async_copy(
                g_hbm.at[src_v.at[j]], msg_v.at[j % NBG], gsem)
        for j in range(ngrp):
            b = j % NBG
            gd[j].wait()
            sd[j] = pltpu.async_copy(
                msg_v.at[b], s_sh.at[dst_v.at[j]], ssem, add=True)
            nj = j + GAH
            if nj < ngrp:
                if nj >= NBG:
                    sd[nj - NBG].wait()
                gd[nj] = pltpu.async_copy(
                    g_hbm.at[src_v.at[nj]], msg_v.at[nj % NBG], gsem)
        for j in range(max(0, ngrp - NBG), ngrp):
            sd[j].wait()
        plsc.subcore_barrier()
        pltpu.sync_copy(s_sh.at[pl.ds(r0, rpt)], sp_hbm.at[cid, pl.ds(r0, rpt)])

    return msg_k


def _encode_body(x_ref, w_ref, degp_ref, g_ref, dr_ref):
    deg = degp_ref[0:1, :] + degp_ref[1:2, :] + 1.0
    dinv_row = lax.rsqrt(deg)
    dinv = jnp.transpose(dinv_row)
    h = jnp.dot(x_ref[...], w_ref[...], preferred_element_type=jnp.float32,
                precision=lax.Precision.HIGHEST)
    g_ref[...] = h * dinv
    dr_ref[...] = dinv_row


def _z_body(sp_ref, g_ref, dr_ref, b_ref, z_ref):
    s = sp_ref[0] + sp_ref[1] + g_ref[...]
    dinv = jnp.transpose(dr_ref[...])
    z_ref[...] = jnp.maximum(dinv * s + b_ref[...], 0.0)


def _decode_body(bmr, bnc, z_ref, out_ref):
    i = pl.program_id(0)
    j = pl.program_id(1)
    zr = z_ref[pl.ds(i * bmr, bmr), :]
    zc = z_ref[pl.ds(j * bnc, bnc), :]
    logits = lax.dot_general(
        zr, zc, (((1,), (1,)), ((), ())),
        preferred_element_type=jnp.float32)
    out_ref[...] = 0.5 * jnp.tanh(0.5 * logits) + 0.5


def kernel(x, edge_index, W, b):
    n, d_in = x.shape
    dh = W.shape[1]
    e = edge_index.shape[1]

    n_pad = ((n + 128 + 255) // 256) * 256
    assert e % (NW * L) == 0
    epw0 = e // NW
    nfull, rem = divmod(epw0, GC)
    ngrp = nfull + (1 if rem else 0)
    ei32 = edge_index.astype(jnp.int32)

    degp = _deg_kernel(ngrp, epw0, nfull, rem, n, n_pad)(ei32)

    bm = n_pad
    g_pad, dinv_row = pl.pallas_call(
        _encode_body,
        grid=(n_pad // bm,),
        in_specs=[
            pl.BlockSpec((bm, d_in), lambda i: (i, 0)),
            pl.BlockSpec((d_in, dh), lambda i: (0, 0)),
            pl.BlockSpec((2, bm), lambda i: (0, i)),
        ],
        out_specs=[
            pl.BlockSpec((bm, dh), lambda i: (i, 0)),
            pl.BlockSpec((1, bm), lambda i: (0, i)),
        ],
        out_shape=[
            jax.ShapeDtypeStruct((n_pad, dh), jnp.float32),
            jax.ShapeDtypeStruct((1, n_pad), jnp.float32),
        ],
    )(x, W, degp)

    zeros2d = jnp.zeros((n_pad, dh), jnp.float32)
    sp = _msg_kernel(ngrp, epw0, nfull, rem, n, n_pad, dh)(g_pad, ei32, zeros2d)

    z_pad = pl.pallas_call(
        _z_body,
        grid=(n_pad // bm,),
        in_specs=[
            pl.BlockSpec((2, bm, dh), lambda i: (0, i, 0)),
            pl.BlockSpec((bm, dh), lambda i: (i, 0)),
            pl.BlockSpec((1, bm), lambda i: (0, i)),
            pl.BlockSpec((1, dh), lambda i: (0, 0)),
        ],
        out_specs=pl.BlockSpec((bm, dh), lambda i: (i, 0)),
        out_shape=jax.ShapeDtypeStruct((n_pad, dh), jnp.float32),
    )(sp, g_pad, dinv_row, b.reshape(1, dh))

    bmr, bnc = 2048, 2048
    adj = pl.pallas_call(
        functools.partial(_decode_body, bmr, bnc),
        grid=(-(-n // bmr), -(-n // bnc)),
        in_specs=[
            pl.BlockSpec((n_pad, dh), lambda i, j: (0, 0)),
        ],
        out_specs=pl.BlockSpec((bmr, bnc), lambda i, j: (i, j)),
        out_shape=jax.ShapeDtypeStruct((n, n), jnp.float32),
    )(z_pad)
    return adj


# speedup vs baseline: 1.0520x; 1.0492x over previous
import functools

import jax
import jax.numpy as jnp
from jax import lax
from jax.experimental import pallas as pl
from jax.experimental.pallas import tpu as pltpu
from jax.experimental.pallas import tpu_sc as plsc

NC = 2
NS = 16
NW = NC * NS
L = 16
CH = 128
GRP = 8
NBG = 5
GAH = 3


def _mesh():
    return plsc.VectorSubcoreMesh(
        core_axis_name="c", subcore_axis_name="s", num_cores=NC, num_subcores=NS)


_SC_PARAMS = pltpu.CompilerParams(use_tc_tiling_on_sc=False)


GC = GRP * CH


def _load_edge_row(ei_hbm, row, idx_v, wid, epw0, nfull, rem, n, lsem):
    descs = []
    base = wid * epw0
    for r in range(nfull):
        descs.append(pltpu.async_copy(
            ei_hbm.at[row, pl.ds(base + r * GC, GC)], idx_v.at[r], lsem))
    if rem:
        iota16 = lax.iota(jnp.int32, L)
        for k in range((GC - rem) // L):
            idx_v[nfull, pl.ds(rem + k * L, L)] = n + ((iota16 + k * L) & 127)
        descs.append(pltpu.async_copy(
            ei_hbm.at[row, pl.ds(base + nfull * GC, rem)],
            idx_v.at[nfull, pl.ds(0, rem)], lsem))
    return descs


def _deg_kernel(ngrp, epw0, nfull, rem, n, n_pad):
    rpt = n_pad // NS

    @functools.partial(
        pl.kernel,
        out_type=jax.ShapeDtypeStruct((NC, n_pad), jnp.float32),
        mesh=_mesh(),
        scratch_types=[
            pltpu.VMEM((ngrp, GC), jnp.int32),
            pltpu.VMEM((GC,), jnp.float32),
            pltpu.VMEM((rpt,), jnp.float32),
            pltpu.VMEM_SHARED((n_pad,), jnp.float32),
            pltpu.SemaphoreType.DMA,
            pltpu.SemaphoreType.DMA,
        ],
        compiler_params=_SC_PARAMS,
    )
    def deg_k(ei_hbm, degp_hbm, idx_v, ones_v, stage_v, deg_sh, sem, lsem):
        cid = lax.axis_index("c")
        tid = lax.axis_index("s")
        wid = cid * NS + tid
        r0 = tid * rpt
        one16 = jnp.ones((L,), jnp.float32)
        zero16 = jnp.zeros((L,), jnp.float32)
        ld = _load_edge_row(ei_hbm, 1, idx_v, wid, epw0, nfull, rem, n, lsem)
        for k in range(GC // L):
            ones_v[pl.ds(k * L, L)] = one16
        for k in range(rpt // L):
            stage_v[pl.ds(k * L, L)] = zero16
        pltpu.sync_copy(stage_v, deg_sh.at[pl.ds(r0, rpt)])
        for dsc in ld:
            dsc.wait()
        plsc.subcore_barrier()
        descs = [None] * ngrp
        for j in range(ngrp):
            descs[j] = pltpu.async_copy(
                ones_v, deg_sh.at[idx_v.at[j]], sem, add=True)
            if j >= 4:
                descs[j - 4].wait()
        for j in range(max(0, ngrp - 4), ngrp):
            descs[j].wait()
        plsc.subcore_barrier()
        pltpu.sync_copy(deg_sh.at[pl.ds(r0, rpt)], degp_hbm.at[cid, pl.ds(r0, rpt)])

    return deg_k


def _msg_kernel(ngrp, epw0, nfull, rem, n, n_pad, dh):
    rpt = n_pad // NS

    @functools.partial(
        pl.kernel,
        out_type=jax.ShapeDtypeStruct((NC, n_pad, dh), jnp.float32),
        mesh=_mesh(),
        scratch_types=[
            pltpu.VMEM((ngrp, GC), jnp.int32),
            pltpu.VMEM((ngrp, GC), jnp.int32),
            pltpu.VMEM((NBG, GC, dh), jnp.float32),
            pltpu.VMEM((rpt, dh), jnp.float32),
            pltpu.VMEM_SHARED((n_pad, dh), jnp.float32),
            pltpu.SemaphoreType.DMA,
            pltpu.SemaphoreType.DMA,
            pltpu.SemaphoreType.DMA,
        ],
        compiler_params=_SC_PARAMS,
    )
    def msg_k(g_hbm, ei_hbm, zero_hbm, sp_hbm,
              src_v, dst_v, msg_v, stage_v, s_sh, gsem, ssem, lsem):
        cid = lax.axis_index("c")
        tid = lax.axis_index("s")
        wid = cid * NS + tid
        r0 = tid * rpt
        ld = _load_edge_row(ei_hbm, 0, src_v, wid, epw0, nfull, rem, n, lsem)
        ld += _load_edge_row(ei_hbm, 1, dst_v, wid, epw0, nfull, rem, n, lsem)
        pltpu.sync_copy(zero_hbm.at[pl.ds(r0, rpt)], s_sh.at[pl.ds(r0, rpt)])
        for dsc in ld:
            dsc.wait()
        plsc.subcore_barrier()
        gd = [None] * ngrp
        sd = [None] * ngrp
        for j in range(min(GAH, ngrp)):
            gd[j] = pltpu.async_copy(
                g_hbm.at[src_v.at[j]], msg_v.at[j % NBG], gsem)
        for j in range(ngrp):
            b = j % NBG
            gd[j].wait()
            sd[j] = pltpu.async_copy(
                msg_v.at[b], s_sh.at[dst_v.at[j]], ssem, add=True)
            nj = j + GAH
            if nj < ngrp:
                if nj >= NBG:
                    sd[nj - NBG].wait()
                gd[nj] = pltpu.async_copy(
                    g_hbm.at[src_v.at[nj]], msg_v.at[nj % NBG], gsem)
        for j in range(max(0, ngrp - NBG), ngrp):
            sd[j].wait()
        plsc.subcore_barrier()
        pltpu.sync_copy(s_sh.at[pl.ds(r0, rpt)], sp_hbm.at[cid, pl.ds(r0, rpt)])

    return msg_k


def _z_kernel(n_pad, dh):
    rpw = n_pad // NW
    flw = rpw * dh

    @functools.partial(
        pl.kernel,
        out_type=jax.ShapeDtypeStruct((n_pad, dh), jnp.float32),
        mesh=_mesh(),
        scratch_types=[
            pltpu.VMEM((rpw, dh), jnp.float32),
            pltpu.VMEM((rpw, dh), jnp.float32),
            pltpu.VMEM((rpw, dh), jnp.float32),
            pltpu.VMEM((rpw,), jnp.float32),
            pltpu.VMEM((dh,), jnp.float32),
            pltpu.VMEM((rpw, dh), jnp.float32),
            pltpu.SemaphoreType.DMA,
        ],
        compiler_params=_SC_PARAMS,
    )
    def z_k(sp_hbm, g_hbm, di_hbm, b_hbm, z_hbm,
            s0_v, s1_v, g_v, di_v, b_v, z_v, lsem):
        cid = lax.axis_index("c")
        tid = lax.axis_index("s")
        wid = cid * NS + tid
        row0 = wid * rpw
        ds = [
            pltpu.async_copy(sp_hbm.at[0, pl.ds(row0, rpw)], s0_v, lsem),
            pltpu.async_copy(sp_hbm.at[1, pl.ds(row0, rpw)], s1_v, lsem),
            pltpu.async_copy(g_hbm.at[pl.ds(row0, rpw)], g_v, lsem),
            pltpu.async_copy(di_hbm.at[0, pl.ds(row0, rpw)], di_v, lsem),
            pltpu.async_copy(b_hbm, b_v, lsem),
        ]
        for d in ds:
            d.wait()
        bvec = b_v[...]
        lane_ids = [jnp.full((L,), k, jnp.int32) for k in range(L)]

        def body(i, carry):
            dv16 = di_v[pl.ds(i * L, L)]
            for k in range(L):
                row = i * L + k
                s = s0_v[row] + s1_v[row] + g_v[row]
                dvb = dv16[lane_ids[k]]
                z_v[row] = jnp.maximum(s * dvb + bvec, 0.0)
            return carry

        lax.fori_loop(0, rpw // L, body, 0)
        pltpu.sync_copy(z_v, z_hbm.at[pl.ds(row0, rpw)])

    return z_k


def _encode_body(x_ref, w_ref, degp_ref, g_ref, dr_ref):
    deg = degp_ref[0:1, :] + degp_ref[1:2, :] + 1.0
    dinv_row = lax.rsqrt(deg)
    dinv = jnp.transpose(dinv_row)
    h = jnp.dot(x_ref[...], w_ref[...], preferred_element_type=jnp.float32,
                precision=lax.Precision.HIGHEST)
    g_ref[...] = h * dinv
    dr_ref[...] = dinv_row


def _z_body(sp_ref, g_ref, dr_ref, b_ref, z_ref):
    s = sp_ref[0] + sp_ref[1] + g_ref[...]
    dinv = jnp.transpose(dr_ref[...])
    z_ref[...] = jnp.maximum(dinv * s + b_ref[...], 0.0)


def _decode_body(bmr, bnc, z_ref, out_ref):
    i = pl.program_id(0)
    j = pl.program_id(1)
    zr = z_ref[pl.ds(i * bmr, bmr), :]
    zc = z_ref[pl.ds(j * bnc, bnc), :]
    logits = lax.dot_general(
        zr, zc, (((1,), (1,)), ((), ())),
        preferred_element_type=jnp.float32)
    out_ref[...] = 0.5 * jnp.tanh(0.5 * logits) + 0.5


def kernel(x, edge_index, W, b):
    n, d_in = x.shape
    dh = W.shape[1]
    e = edge_index.shape[1]

    n_pad = ((n + 128 + 255) // 256) * 256
    assert e % (NW * L) == 0
    epw0 = e // NW
    nfull, rem = divmod(epw0, GC)
    ngrp = nfull + (1 if rem else 0)
    ei32 = edge_index.astype(jnp.int32)

    degp = _deg_kernel(ngrp, epw0, nfull, rem, n, n_pad)(ei32)

    bm = n_pad
    g_pad, dinv_row = pl.pallas_call(
        _encode_body,
        grid=(n_pad // bm,),
        in_specs=[
            pl.BlockSpec((bm, d_in), lambda i: (i, 0)),
            pl.BlockSpec((d_in, dh), lambda i: (0, 0)),
            pl.BlockSpec((2, bm), lambda i: (0, i)),
        ],
        out_specs=[
            pl.BlockSpec((bm, dh), lambda i: (i, 0)),
            pl.BlockSpec((1, bm), lambda i: (0, i)),
        ],
        out_shape=[
            jax.ShapeDtypeStruct((n_pad, dh), jnp.float32),
            jax.ShapeDtypeStruct((1, n_pad), jnp.float32),
        ],
    )(x, W, degp)

    zeros2d = jnp.zeros((n_pad, dh), jnp.float32)
    sp = _msg_kernel(ngrp, epw0, nfull, rem, n, n_pad, dh)(g_pad, ei32, zeros2d)

    z_pad = _z_kernel(n_pad, dh)(sp, g_pad, dinv_row, b)

    bmr, bnc = 2048, 2048
    adj = pl.pallas_call(
        functools.partial(_decode_body, bmr, bnc),
        grid=(-(-n // bmr), -(-n // bnc)),
        in_specs=[
            pl.BlockSpec((n_pad, dh), lambda i, j: (0, 0)),
        ],
        out_specs=pl.BlockSpec((bmr, bnc), lambda i, j: (i, j)),
        out_shape=jax.ShapeDtypeStruct((n, n), jnp.float32),
    )(z_pad)
    return adj
